# baseline (speedup 1.0000x reference)
import jax
import jax.numpy as jnp
import numpy as np
from jax import lax
from jax.experimental import pallas as pl
from jax.experimental.pallas import tpu as pltpu
from jax.experimental.pallas import tpu_sc as plsc

N = 10000
E = 320000
H = 128
NGRAPH = 64
NLAYER = 3
BN_EPS = 1e-5

C = 80
NW = 32
EPT = E // NW
NCH = EPT // C
NPAIR = (NCH - 1) // 2
NPAD = 10240
RPT = NPAD // 16


def _sc_body(h_hbm, ea_hbm, src_hbm, dst_hbm, out_hbm,
             agg_sh, s0, s1, d0, d1, rows0, ea0, rows1, ea1,
             gsem0, gsem1, ssem0, ssem1, dsem0, dsem1):
    c = lax.axis_index("c")
    s = lax.axis_index("s")
    wid = s * 2 + c
    ebase = wid * EPT

    zero = jnp.zeros((16,), jnp.float32)

    def zrow(e, carry):
        for v in range(H // 16):
            rows0[e, pl.ds(v * 16, 16)] = zero
        return carry

    lax.fori_loop(0, C, zrow, 0)
    for t in range(RPT // C):
        pltpu.sync_copy(rows0, agg_sh.at[pl.ds(s * RPT + t * C, C)])
    plsc.subcore_barrier()

    sbuf, dbuf = (s0, s1), (d0, d1)
    rbuf, ebuf = (rows0, rows1), (ea0, ea1)
    gsem, ssem, dsem = (gsem0, gsem1), (ssem0, ssem1), (dsem0, dsem1)

    def ioff(k):
        return ebase + jnp.minimum(k, NCH - 1) * C

    def idx_s_start(k, b):
        pltpu.async_copy(src_hbm.at[pl.ds(ioff(k), C)], sbuf[b], ssem[b])

    def idx_s_wait(k, b):
        pltpu.make_async_copy(src_hbm.at[pl.ds(ioff(k), C)],
                              sbuf[b], ssem[b]).wait()

    def idx_d_start(k, b):
        pltpu.async_copy(dst_hbm.at[pl.ds(ioff(k), C)], dbuf[b], dsem[b])

    def idx_d_wait(k, b):
        pltpu.make_async_copy(dst_hbm.at[pl.ds(ioff(k), C)],
                              dbuf[b], dsem[b]).wait()

    def g_start(k, b):
        pltpu.async_copy(h_hbm.at[sbuf[b]], rbuf[b], gsem[b])
        pltpu.async_copy(ea_hbm.at[pl.ds(ebase + k * C, C)], ebuf[b], gsem[b])

    def g_wait(k, b):
        pltpu.make_async_copy(h_hbm.at[sbuf[b]], rbuf[b], gsem[b]).wait()
        pltpu.make_async_copy(ea_hbm.at[pl.ds(ebase + k * C, C)],
                              ebuf[b], gsem[b]).wait()

    def compute(b):
        rows, ea = rbuf[b], ebuf[b]

        def blk(t, tcarry):
            sl = pl.ds(t * 8, 8)
            rows[sl, :] = jnp.maximum(
                rows[sl, :] + ea[sl, :].astype(jnp.float32), 0.0)
            return tcarry

        lax.fori_loop(0, C // 8, blk, 0)

    def scat(b):
        pltpu.sync_copy(rbuf[b], agg_sh.at[dbuf[b]], add=True)

    def half(k, b):
        idx_s_wait(k + 1, 1 - b)
        g_start(k + 1, 1 - b)
        g_wait(k, b)
        idx_s_start(k + 2, b)
        idx_d_wait(k, b)
        compute(b)
        scat(b)
        idx_d_start(k + 2, b)

    pltpu.sync_copy(src_hbm.at[pl.ds(ebase, C)], s0)
    g_start(0, 0)
    idx_s_start(1, 1)
    idx_d_start(0, 0)
    idx_d_start(1, 1)

    def pair(j, carry):
        k0 = 2 * j
        half(k0, 0)
        half(k0 + 1, 1)
        return carry

    lax.fori_loop(0, NPAIR, pair, 0)
    g_wait(NCH - 1, 0)
    idx_d_wait(NCH - 1, 0)
    compute(0)
    scat(0)
    idx_s_wait(NCH + 1, 1)
    idx_d_wait(NCH + 1, 1)
    plsc.subcore_barrier()

    for t in range(RPT // C):
        off = s * RPT + t * C
        pltpu.sync_copy(agg_sh.at[pl.ds(off, C)], rows0)
        pltpu.sync_copy(rows0, out_hbm.at[pl.ds(c * NPAD + off, C)])


_SC_CACHE = {}


def _sc_gather_scatter(h, ea, src, dst):
    if "k" not in _SC_CACHE:
        _SC_CACHE["k"] = pl.kernel(
            _sc_body,
            out_type=jax.ShapeDtypeStruct((2 * NPAD, H), jnp.float32),
            mesh=plsc.VectorSubcoreMesh(core_axis_name="c",
                                        subcore_axis_name="s"),
            scratch_types=[
                pltpu.VMEM_SHARED((NPAD, H), jnp.float32),
                pltpu.VMEM((C,), jnp.int32),
                pltpu.VMEM((C,), jnp.int32),
                pltpu.VMEM((C,), jnp.int32),
                pltpu.VMEM((C,), jnp.int32),
                pltpu.VMEM((C, H), jnp.float32),
                pltpu.VMEM((C, H), jnp.bfloat16),
                pltpu.VMEM((C, H), jnp.float32),
                pltpu.VMEM((C, H), jnp.bfloat16),
                pltpu.SemaphoreType.DMA,
                pltpu.SemaphoreType.DMA,
                pltpu.SemaphoreType.DMA,
                pltpu.SemaphoreType.DMA,
                pltpu.SemaphoreType.DMA,
                pltpu.SemaphoreType.DMA,
            ],
        )
    return _SC_CACHE["k"](h, ea, src, dst)


def _enc_body(x_ref, w_ref, b_ref, o_ref):
    o_ref[...] = (jnp.dot(x_ref[...], w_ref[...],
                          preferred_element_type=jnp.float32) + b_ref[...])


def _enc_bf16_body(x_ref, w_ref, b_ref, o_ref):
    o_ref[...] = (jnp.dot(x_ref[...], w_ref[...],
                          preferred_element_type=jnp.float32)
                  + b_ref[...]).astype(jnp.bfloat16)


def _make_enc(rows, block, in_dim, body=_enc_body, out_dtype=jnp.float32,
              out_w=H):
    return pl.pallas_call(
        body,
        grid=(rows // block,),
        in_specs=[
            pl.BlockSpec((block, in_dim), lambda i: (i, 0)),
            pl.BlockSpec((in_dim, H), lambda i: (0, 0)),
            pl.BlockSpec((1, H), lambda i: (0, 0)),
        ],
        out_specs=pl.BlockSpec((block, out_w), lambda i: (i, 0)),
        out_shape=jax.ShapeDtypeStruct((rows, out_w), out_dtype),
    )


_node_enc = _make_enc(N, 2000, H)
_edge_enc = _make_enc(E, 4000, 16, body=_enc_bf16_body,
                      out_dtype=jnp.bfloat16)

_MB = 2000


def _mlp_body(h_ref, agg_ref, w1_ref, b1_ref, w2_ref, b2_ref, o_ref):
    z = h_ref[...] + agg_ref[0] + agg_ref[1]
    z = jnp.dot(z, w1_ref[...], preferred_element_type=jnp.float32) + b1_ref[...]
    z = jnp.maximum(z, 0.0)
    z = jnp.dot(z, w2_ref[...], preferred_element_type=jnp.float32) + b2_ref[...]
    o_ref[...] = jnp.maximum(z, 0.0)


_mlp = pl.pallas_call(
    _mlp_body,
    grid=(N // _MB,),
    in_specs=[
        pl.BlockSpec((_MB, H), lambda i: (i, 0)),
        pl.BlockSpec((2, _MB, H), lambda i: (0, i, 0)),
        pl.BlockSpec((H, H), lambda i: (0, 0)),
        pl.BlockSpec((1, H), lambda i: (0, 0)),
        pl.BlockSpec((H, H), lambda i: (0, 0)),
        pl.BlockSpec((1, H), lambda i: (0, 0)),
    ],
    out_specs=pl.BlockSpec((_MB, H), lambda i: (i, 0)),
    out_shape=jax.ShapeDtypeStruct((N, H), jnp.float32),
)


def _pool_body(h_ref, batch_ref, wfc_ref, bfc_ref, o_ref, acc_ref):
    i = pl.program_id(0)
    seg = batch_ref[0, 0, :]
    iot = lax.broadcasted_iota(jnp.int32, (NGRAPH, _MB), 0)
    oh = (iot == seg[None, :]).astype(jnp.float32)
    part = jnp.dot(oh, h_ref[...], preferred_element_type=jnp.float32)

    @pl.when(i == 0)
    def _():
        acc_ref[...] = part

    @pl.when(i > 0)
    def _():
        acc_ref[...] = acc_ref[...] + part

    @pl.when(i == N // _MB - 1)
    def _():
        o_ref[...] = (jnp.dot(acc_ref[...], wfc_ref[...],
                              preferred_element_type=jnp.float32) + bfc_ref[...])


_pool = pl.pallas_call(
    _pool_body,
    grid=(N // _MB,),
    in_specs=[
        pl.BlockSpec((_MB, H), lambda i: (i, 0)),
        pl.BlockSpec((1, 1, _MB), lambda i: (i, 0, 0)),
        pl.BlockSpec((H, H), lambda i: (0, 0)),
        pl.BlockSpec((1, H), lambda i: (0, 0)),
    ],
    out_specs=pl.BlockSpec((NGRAPH, H), lambda i: (0, 0)),
    out_shape=jax.ShapeDtypeStruct((NGRAPH, H), jnp.float32),
    scratch_shapes=[pltpu.VMEM((NGRAPH, H), jnp.float32)],
)


def kernel(x, edge_index, edge_attr, batch, W_node, b_node, W_edge, b_edge,
           W1, b1, gamma, beta, W2, b2, W_fc, b_fc):
    src = edge_index[0].astype(jnp.int32)
    dst = edge_index[1].astype(jnp.int32)
    scale = gamma / jnp.sqrt(1.0 + BN_EPS)
    W1f = W1 * scale[:, None, :]
    b1f = b1 * scale + beta

    h = _node_enc(x, W_node, b_node.reshape(1, H))
    ea = _edge_enc(edge_attr, W_edge, b_edge.reshape(1, H))
    batch3 = batch.astype(jnp.int32).reshape(N // _MB, 1, _MB)

    for i in range(NLAYER):
        agg = _sc_gather_scatter(h, ea, src, dst).reshape(2, NPAD, H)
        h = _mlp(h, agg, W1f[i], b1f[i].reshape(1, H),
                 W2[i], b2[i].reshape(1, H))

    return _pool(h, batch3, W_fc, b_fc.reshape(1, H))


# speedup vs baseline: 1.0880x; 1.0880x over previous
import jax
import jax.numpy as jnp
import numpy as np
from jax import lax
from jax.experimental import pallas as pl
from jax.experimental.pallas import tpu as pltpu
from jax.experimental.pallas import tpu_sc as plsc

N = 10000
E = 320000
H = 128
NGRAPH = 64
NLAYER = 3
BN_EPS = 1e-5

C = 80
NW = 32
EPT = E // NW
NCH = EPT // C
NPAIR = (NCH - 1) // 2
NPAD = 10240
RPT = NPAD // 16


def _sc_body(h_hbm, ea_hbm, src_hbm, dst_hbm, out_hbm,
             agg_sh, s0, s1, s2, d0, d1, d2,
             rows0, ea0, rows1, ea1, rows2, ea2,
             gsem0, gsem1, gsem2, osem0, osem1, osem2,
             ssem0, ssem1, ssem2, dsem0, dsem1, dsem2):
    c = lax.axis_index("c")
    s = lax.axis_index("s")
    wid = s * 2 + c
    ebase = wid * EPT

    zero = jnp.zeros((16,), jnp.float32)

    def zrow(e, carry):
        for v in range(H // 16):
            rows0[e, pl.ds(v * 16, 16)] = zero
        return carry

    lax.fori_loop(0, C, zrow, 0)
    for t in range(RPT // C):
        pltpu.sync_copy(rows0, agg_sh.at[pl.ds(s * RPT + t * C, C)])
    plsc.subcore_barrier()

    sbuf, dbuf = (s0, s1, s2), (d0, d1, d2)
    rbuf, ebuf = (rows0, rows1, rows2), (ea0, ea1, ea2)
    gsem, osem = (gsem0, gsem1, gsem2), (osem0, osem1, osem2)
    ssem, dsem = (ssem0, ssem1, ssem2), (dsem0, dsem1, dsem2)

    def ioff(k):
        return ebase + jnp.minimum(k, NCH - 1) * C

    def idx_s_start(k, b):
        pltpu.async_copy(src_hbm.at[pl.ds(ioff(k), C)], sbuf[b], ssem[b])

    def idx_s_wait(k, b):
        pltpu.make_async_copy(src_hbm.at[pl.ds(ioff(k), C)],
                              sbuf[b], ssem[b]).wait()

    def idx_d_start(k, b):
        pltpu.async_copy(dst_hbm.at[pl.ds(ioff(k), C)], dbuf[b], dsem[b])

    def idx_d_wait(k, b):
        pltpu.make_async_copy(dst_hbm.at[pl.ds(ioff(k), C)],
                              dbuf[b], dsem[b]).wait()

    def g_start(k, b):
        pltpu.async_copy(h_hbm.at[sbuf[b]], rbuf[b], gsem[b])
        pltpu.async_copy(ea_hbm.at[pl.ds(ebase + k * C, C)], ebuf[b], gsem[b])

    def g_wait(k, b):
        pltpu.make_async_copy(h_hbm.at[sbuf[b]], rbuf[b], gsem[b]).wait()
        pltpu.make_async_copy(ea_hbm.at[pl.ds(ebase + k * C, C)],
                              ebuf[b], gsem[b]).wait()

    def compute(b):
        rows, ea = rbuf[b], ebuf[b]

        def blk(t, tcarry):
            sl = pl.ds(t * 8, 8)
            rows[sl, :] = jnp.maximum(
                rows[sl, :] + ea[sl, :].astype(jnp.float32), 0.0)
            return tcarry

        lax.fori_loop(0, C // 8, blk, 0)

    def scat_start(b):
        pltpu.async_copy(rbuf[b], agg_sh.at[dbuf[b]], osem[b], add=True)

    def scat_wait(b):
        pltpu.make_async_copy(rbuf[b], agg_sh.at[dbuf[b]], osem[b]).wait()

    def steady(k, b):
        b2 = (b + 2) % 3
        g_wait(k, b)
        scat_wait(b2)
        idx_s_wait(k + 2, b2)
        idx_d_start(k + 2, b2)
        g_start(k + 2, b2)
        idx_s_start(k + 3, b)
        idx_d_wait(k, b)
        compute(b)
        scat_start(b)

    pltpu.sync_copy(src_hbm.at[pl.ds(ebase, C)], s0)
    pltpu.sync_copy(src_hbm.at[pl.ds(ebase + C, C)], s1)
    g_start(0, 0)
    g_start(1, 1)
    idx_s_start(2, 2)
    idx_d_start(0, 0)
    idx_d_start(1, 1)
    g_wait(0, 0)
    idx_s_wait(2, 2)
    idx_d_start(2, 2)
    g_start(2, 2)
    idx_s_start(3, 0)
    idx_d_wait(0, 0)
    compute(0)
    scat_start(0)

    def triple(j, carry):
        k = 3 * j + 1
        steady(k, 1)
        steady(k + 1, 2)
        steady(k + 2, 0)
        return carry

    lax.fori_loop(0, (NCH - 2) // 3, triple, 0)
    g_wait(NCH - 1, 1)
    scat_wait(0)
    idx_d_wait(NCH - 1, 1)
    compute(1)
    scat_start(1)
    scat_wait(1)
    idx_s_wait(NCH + 1, 0)
    idx_d_wait(NCH, 2)
    plsc.subcore_barrier()

    for t in range(RPT // C):
        off = s * RPT + t * C
        pltpu.sync_copy(agg_sh.at[pl.ds(off, C)], rows0)
        pltpu.sync_copy(rows0, out_hbm.at[pl.ds(c * NPAD + off, C)])


_SC_CACHE = {}


def _sc_gather_scatter(h, ea, src, dst):
    if "k" not in _SC_CACHE:
        _SC_CACHE["k"] = pl.kernel(
            _sc_body,
            out_type=jax.ShapeDtypeStruct((2 * NPAD, H), jnp.float32),
            mesh=plsc.VectorSubcoreMesh(core_axis_name="c",
                                        subcore_axis_name="s"),
            scratch_types=(
                [pltpu.VMEM_SHARED((NPAD, H), jnp.float32)]
                + [pltpu.VMEM((C,), jnp.int32)] * 6
                + [pltpu.VMEM((C, H), jnp.float32),
                   pltpu.VMEM((C, H), jnp.bfloat16)] * 3
                + [pltpu.SemaphoreType.DMA] * 12
            ),
        )
    return _SC_CACHE["k"](h, ea, src, dst)


def _enc_body(x_ref, w_ref, b_ref, o_ref):
    o_ref[...] = (jnp.dot(x_ref[...], w_ref[...],
                          preferred_element_type=jnp.float32) + b_ref[...])


def _enc_bf16_body(x_ref, w_ref, b_ref, o_ref):
    o_ref[...] = (jnp.dot(x_ref[...], w_ref[...],
                          preferred_element_type=jnp.float32)
                  + b_ref[...]).astype(jnp.bfloat16)


def _make_enc(rows, block, in_dim, body=_enc_body, out_dtype=jnp.float32,
              out_w=H):
    return pl.pallas_call(
        body,
        grid=(rows // block,),
        in_specs=[
            pl.BlockSpec((block, in_dim), lambda i: (i, 0)),
            pl.BlockSpec((in_dim, H), lambda i: (0, 0)),
            pl.BlockSpec((1, H), lambda i: (0, 0)),
        ],
        out_specs=pl.BlockSpec((block, out_w), lambda i: (i, 0)),
        out_shape=jax.ShapeDtypeStruct((rows, out_w), out_dtype),
    )


_node_enc = _make_enc(N, 2000, H)
_edge_enc = _make_enc(E, 4000, 16, body=_enc_bf16_body,
                      out_dtype=jnp.bfloat16)

_MB = 2000


def _mlp_body(h_ref, agg_ref, w1_ref, b1_ref, w2_ref, b2_ref, o_ref):
    z = h_ref[...] + agg_ref[0] + agg_ref[1]
    z = jnp.dot(z, w1_ref[...], preferred_element_type=jnp.float32) + b1_ref[...]
    z = jnp.maximum(z, 0.0)
    z = jnp.dot(z, w2_ref[...], preferred_element_type=jnp.float32) + b2_ref[...]
    o_ref[...] = jnp.maximum(z, 0.0)


_mlp = pl.pallas_call(
    _mlp_body,
    grid=(N // _MB,),
    in_specs=[
        pl.BlockSpec((_MB, H), lambda i: (i, 0)),
        pl.BlockSpec((2, _MB, H), lambda i: (0, i, 0)),
        pl.BlockSpec((H, H), lambda i: (0, 0)),
        pl.BlockSpec((1, H), lambda i: (0, 0)),
        pl.BlockSpec((H, H), lambda i: (0, 0)),
        pl.BlockSpec((1, H), lambda i: (0, 0)),
    ],
    out_specs=pl.BlockSpec((_MB, H), lambda i: (i, 0)),
    out_shape=jax.ShapeDtypeStruct((N, H), jnp.float32),
)


def _pool_body(h_ref, batch_ref, wfc_ref, bfc_ref, o_ref, acc_ref):
    i = pl.program_id(0)
    seg = batch_ref[0, 0, :]
    iot = lax.broadcasted_iota(jnp.int32, (NGRAPH, _MB), 0)
    oh = (iot == seg[None, :]).astype(jnp.float32)
    part = jnp.dot(oh, h_ref[...], preferred_element_type=jnp.float32)

    @pl.when(i == 0)
    def _():
        acc_ref[...] = part

    @pl.when(i > 0)
    def _():
        acc_ref[...] = acc_ref[...] + part

    @pl.when(i == N // _MB - 1)
    def _():
        o_ref[...] = (jnp.dot(acc_ref[...], wfc_ref[...],
                              preferred_element_type=jnp.float32) + bfc_ref[...])


_pool = pl.pallas_call(
    _pool_body,
    grid=(N // _MB,),
    in_specs=[
        pl.BlockSpec((_MB, H), lambda i: (i, 0)),
        pl.BlockSpec((1, 1, _MB), lambda i: (i, 0, 0)),
        pl.BlockSpec((H, H), lambda i: (0, 0)),
        pl.BlockSpec((1, H), lambda i: (0, 0)),
    ],
    out_specs=pl.BlockSpec((NGRAPH, H), lambda i: (0, 0)),
    out_shape=jax.ShapeDtypeStruct((NGRAPH, H), jnp.float32),
    scratch_shapes=[pltpu.VMEM((NGRAPH, H), jnp.float32)],
)


def kernel(x, edge_index, edge_attr, batch, W_node, b_node, W_edge, b_edge,
           W1, b1, gamma, beta, W2, b2, W_fc, b_fc):
    src = edge_index[0].astype(jnp.int32)
    dst = edge_index[1].astype(jnp.int32)
    scale = gamma / jnp.sqrt(1.0 + BN_EPS)
    W1f = W1 * scale[:, None, :]
    b1f = b1 * scale + beta

    h = _node_enc(x, W_node, b_node.reshape(1, H))
    ea = _edge_enc(edge_attr, W_edge, b_edge.reshape(1, H))
    batch3 = batch.astype(jnp.int32).reshape(N // _MB, 1, _MB)

    for i in range(NLAYER):
        agg = _sc_gather_scatter(h, ea, src, dst).reshape(2, NPAD, H)
        h = _mlp(h, agg, W1f[i], b1f[i].reshape(1, H),
                 W2[i], b2[i].reshape(1, H))

    return _pool(h, batch3, W_fc, b_fc.reshape(1, H))


# speedup vs baseline: 1.1008x; 1.0117x over previous
import jax
import jax.numpy as jnp
import numpy as np
from jax import lax
from jax.experimental import pallas as pl
from jax.experimental.pallas import tpu as pltpu
from jax.experimental.pallas import tpu_sc as plsc

N = 10000
E = 320000
H = 128
NGRAPH = 64
NLAYER = 3
BN_EPS = 1e-5

C = 80
NW = 32
EPT = E // NW
NCH = EPT // C
NPAIR = (NCH - 1) // 2
NPAD = 10240
RPT = NPAD // 16


def _sc_body(h_hbm, ea_hbm, src_hbm, dst_hbm, out_hbm,
             agg_sh, s0, s1, s2, d0, d1, d2,
             rows0, ea0, rows1, ea1, rows2, ea2,
             gsem0, gsem1, gsem2, osem0, osem1, osem2,
             ssem0, ssem1, ssem2, dsem0, dsem1, dsem2):
    c = lax.axis_index("c")
    s = lax.axis_index("s")
    wid = s * 2 + c
    ebase = wid * EPT

    sbuf, dbuf = (s0, s1, s2), (d0, d1, d2)
    rbuf, ebuf = (rows0, rows1, rows2), (ea0, ea1, ea2)
    gsem, osem = (gsem0, gsem1, gsem2), (osem0, osem1, osem2)
    ssem, dsem = (ssem0, ssem1, ssem2), (dsem0, dsem1, dsem2)

    def ioff(k):
        return ebase + jnp.minimum(k, NCH - 1) * C

    def idx_s_start(k, b):
        pltpu.async_copy(src_hbm.at[pl.ds(ioff(k), C)], sbuf[b], ssem[b])

    def idx_s_wait(k, b):
        pltpu.make_async_copy(src_hbm.at[pl.ds(ioff(k), C)],
                              sbuf[b], ssem[b]).wait()

    def idx_d_start(k, b):
        pltpu.async_copy(dst_hbm.at[pl.ds(ioff(k), C)], dbuf[b], dsem[b])

    def idx_d_wait(k, b):
        pltpu.make_async_copy(dst_hbm.at[pl.ds(ioff(k), C)],
                              dbuf[b], dsem[b]).wait()

    def g_start(k, b):
        pltpu.async_copy(h_hbm.at[sbuf[b]], rbuf[b], gsem[b])
        pltpu.async_copy(ea_hbm.at[pl.ds(ebase + k * C, C)], ebuf[b], gsem[b])

    def g_wait(k, b):
        pltpu.make_async_copy(h_hbm.at[sbuf[b]], rbuf[b], gsem[b]).wait()
        pltpu.make_async_copy(ea_hbm.at[pl.ds(ebase + k * C, C)],
                              ebuf[b], gsem[b]).wait()

    def compute(b):
        rows, ea = rbuf[b], ebuf[b]

        def blk(t, tcarry):
            sl = pl.ds(t * 8, 8)
            rows[sl, :] = jnp.maximum(
                rows[sl, :] + ea[sl, :].astype(jnp.float32), 0.0)
            return tcarry

        lax.fori_loop(0, C // 8, blk, 0)

    def scat_start(b):
        pltpu.async_copy(rbuf[b], agg_sh.at[dbuf[b]], osem[b], add=True)

    def scat_wait(b):
        pltpu.make_async_copy(rbuf[b], agg_sh.at[dbuf[b]], osem[b]).wait()

    def steady(k, b):
        b2 = (b + 2) % 3
        g_wait(k, b)
        scat_wait(b2)
        idx_s_wait(k + 2, b2)
        idx_d_start(k + 2, b2)
        g_start(k + 2, b2)
        idx_s_start(k + 3, b)
        idx_d_wait(k, b)
        compute(b)
        scat_start(b)

    pltpu.sync_copy(src_hbm.at[pl.ds(ebase, C)], s0)
    pltpu.sync_copy(src_hbm.at[pl.ds(ebase + C, C)], s1)
    g_start(0, 0)
    g_start(1, 1)
    idx_s_start(2, 2)
    idx_d_start(0, 0)
    idx_d_start(1, 1)

    zero = jnp.zeros((16,), jnp.float32)

    def zrow(e, carry):
        for v in range(H // 16):
            rows2[e, pl.ds(v * 16, 16)] = zero
        return carry

    lax.fori_loop(0, C, zrow, 0)
    for t in range(RPT // C):
        pltpu.sync_copy(rows2, agg_sh.at[pl.ds(s * RPT + t * C, C)])
    plsc.subcore_barrier()

    g_wait(0, 0)
    idx_s_wait(2, 2)
    idx_d_start(2, 2)
    g_start(2, 2)
    idx_s_start(3, 0)
    idx_d_wait(0, 0)
    compute(0)
    scat_start(0)

    def triple(j, carry):
        k = 3 * j + 1
        steady(k, 1)
        steady(k + 1, 2)
        steady(k + 2, 0)
        return carry

    lax.fori_loop(0, (NCH - 2) // 3, triple, 0)
    g_wait(NCH - 1, 1)
    scat_wait(0)
    idx_d_wait(NCH - 1, 1)
    compute(1)
    scat_start(1)
    scat_wait(1)
    idx_s_wait(NCH + 1, 0)
    idx_d_wait(NCH, 2)
    plsc.subcore_barrier()

    pltpu.sync_copy(agg_sh.at[pl.ds(s * RPT, RPT)],
                    out_hbm.at[pl.ds(c * NPAD + s * RPT, RPT)])


_SC_CACHE = {}


def _sc_gather_scatter(h, ea, src, dst):
    if "k" not in _SC_CACHE:
        _SC_CACHE["k"] = pl.kernel(
            _sc_body,
            out_type=jax.ShapeDtypeStruct((2 * NPAD, H), jnp.float32),
            mesh=plsc.VectorSubcoreMesh(core_axis_name="c",
                                        subcore_axis_name="s"),
            scratch_types=(
                [pltpu.VMEM_SHARED((NPAD, H), jnp.float32)]
                + [pltpu.VMEM((C,), jnp.int32)] * 6
                + [pltpu.VMEM((C, H), jnp.float32),
                   pltpu.VMEM((C, H), jnp.bfloat16)] * 3
                + [pltpu.SemaphoreType.DMA] * 12
            ),
        )
    return _SC_CACHE["k"](h, ea, src, dst)


def _enc_body(x_ref, w_ref, b_ref, o_ref):
    o_ref[...] = (jnp.dot(x_ref[...], w_ref[...],
                          preferred_element_type=jnp.float32) + b_ref[...])


def _enc_bf16_body(x_ref, w_ref, b_ref, o_ref):
    o_ref[...] = (jnp.dot(x_ref[...], w_ref[...],
                          preferred_element_type=jnp.float32)
                  + b_ref[...]).astype(jnp.bfloat16)


def _make_enc(rows, block, in_dim, body=_enc_body, out_dtype=jnp.float32,
              out_w=H):
    return pl.pallas_call(
        body,
        grid=(rows // block,),
        in_specs=[
            pl.BlockSpec((block, in_dim), lambda i: (i, 0)),
            pl.BlockSpec((in_dim, H), lambda i: (0, 0)),
            pl.BlockSpec((1, H), lambda i: (0, 0)),
        ],
        out_specs=pl.BlockSpec((block, out_w), lambda i: (i, 0)),
        out_shape=jax.ShapeDtypeStruct((rows, out_w), out_dtype),
    )


_node_enc = _make_enc(N, 2000, H)
_edge_enc = _make_enc(E, 4000, 16, body=_enc_bf16_body,
                      out_dtype=jnp.bfloat16)

_MB = 2000


def _mlp_body(h_ref, agg_ref, w1_ref, b1_ref, w2_ref, b2_ref, o_ref):
    z = h_ref[...] + agg_ref[0] + agg_ref[1]
    z = jnp.dot(z, w1_ref[...], preferred_element_type=jnp.float32) + b1_ref[...]
    z = jnp.maximum(z, 0.0)
    z = jnp.dot(z, w2_ref[...], preferred_element_type=jnp.float32) + b2_ref[...]
    o_ref[...] = jnp.maximum(z, 0.0)


_mlp = pl.pallas_call(
    _mlp_body,
    grid=(N // _MB,),
    in_specs=[
        pl.BlockSpec((_MB, H), lambda i: (i, 0)),
        pl.BlockSpec((2, _MB, H), lambda i: (0, i, 0)),
        pl.BlockSpec((H, H), lambda i: (0, 0)),
        pl.BlockSpec((1, H), lambda i: (0, 0)),
        pl.BlockSpec((H, H), lambda i: (0, 0)),
        pl.BlockSpec((1, H), lambda i: (0, 0)),
    ],
    out_specs=pl.BlockSpec((_MB, H), lambda i: (i, 0)),
    out_shape=jax.ShapeDtypeStruct((N, H), jnp.float32),
)


def _pool_body(h_ref, batch_ref, wfc_ref, bfc_ref, o_ref, acc_ref):
    i = pl.program_id(0)
    seg = batch_ref[0, 0, :]
    iot = lax.broadcasted_iota(jnp.int32, (NGRAPH, _MB), 0)
    oh = (iot == seg[None, :]).astype(jnp.float32)
    part = jnp.dot(oh, h_ref[...], preferred_element_type=jnp.float32)

    @pl.when(i == 0)
    def _():
        acc_ref[...] = part

    @pl.when(i > 0)
    def _():
        acc_ref[...] = acc_ref[...] + part

    @pl.when(i == N // _MB - 1)
    def _():
        o_ref[...] = (jnp.dot(acc_ref[...], wfc_ref[...],
                              preferred_element_type=jnp.float32) + bfc_ref[...])


_pool = pl.pallas_call(
    _pool_body,
    grid=(N // _MB,),
    in_specs=[
        pl.BlockSpec((_MB, H), lambda i: (i, 0)),
        pl.BlockSpec((1, 1, _MB), lambda i: (i, 0, 0)),
        pl.BlockSpec((H, H), lambda i: (0, 0)),
        pl.BlockSpec((1, H), lambda i: (0, 0)),
    ],
    out_specs=pl.BlockSpec((NGRAPH, H), lambda i: (0, 0)),
    out_shape=jax.ShapeDtypeStruct((NGRAPH, H), jnp.float32),
    scratch_shapes=[pltpu.VMEM((NGRAPH, H), jnp.float32)],
)


def kernel(x, edge_index, edge_attr, batch, W_node, b_node, W_edge, b_edge,
           W1, b1, gamma, beta, W2, b2, W_fc, b_fc):
    src = edge_index[0].astype(jnp.int32)
    dst = edge_index[1].astype(jnp.int32)
    scale = gamma / jnp.sqrt(1.0 + BN_EPS)
    W1f = W1 * scale[:, None, :]
    b1f = b1 * scale + beta

    h = _node_enc(x, W_node, b_node.reshape(1, H))
    ea = _edge_enc(edge_attr, W_edge, b_edge.reshape(1, H))
    batch3 = batch.astype(jnp.int32).reshape(N // _MB, 1, _MB)

    for i in range(NLAYER):
        agg = _sc_gather_scatter(h, ea, src, dst).reshape(2, NPAD, H)
        h = _mlp(h, agg, W1f[i], b1f[i].reshape(1, H),
                 W2[i], b2[i].reshape(1, H))

    return _pool(h, batch3, W_fc, b_fc.reshape(1, H))
